# Initial kernel scaffold; baseline (speedup 1.0000x reference)
import jax
import jax.numpy as jnp
from jax.experimental import pallas as pl


def kernel(h, edge_index, w, d, W_w, W_b):
    raise NotImplementedError("write your pallas kernel here")


# speedup vs baseline: 14.5851x; 14.5851x over previous
import functools

import jax
import jax.numpy as jnp
from jax import lax
from jax.experimental import pallas as pl
from jax.experimental.pallas import tpu as pltpu
from jax.experimental.pallas import tpu_sc as plsc

NC = 2
NS = 16
NW = NC * NS
LANES = 16

CHUNK = 128


_SPLAT_DN = lax.GatherDimensionNumbers(
    offset_dims=(), collapsed_slice_dims=(0,), start_index_map=(0,))


def _splat(vec, k):
    idx = jnp.full((LANES, 1), k, jnp.int32)
    return lax.gather(vec, idx, _SPLAT_DN, (1,),
                      mode=lax.GatherScatterMode.PROMISE_IN_BOUNDS)


def _matmul_body(h_ref, wt_ref, b_ref, d_ref, g_ref):
    acc = jnp.dot(h_ref[...], wt_ref[...], preferred_element_type=jnp.float32)
    g_ref[...] = (acc + b_ref[...]) * d_ref[...]


def _combine_body(p_ref, d_ref, z_ref):
    z_ref[...] = d_ref[...] * (p_ref[0] + p_ref[1])


def _sc_aggregate(n_nodes_pad, d_feat, n_chunks):
    mesh = plsc.VectorSubcoreMesh(
        core_axis_name="c", subcore_axis_name="s", num_cores=NC, num_subcores=NS
    )
    rows_per_tile = n_nodes_pad // NS

    @functools.partial(
        pl.kernel,
        out_type=jax.ShapeDtypeStruct((NC, n_nodes_pad, d_feat), jnp.float32),
        mesh=mesh,
        scratch_types=[
            pltpu.VMEM((n_chunks, CHUNK), jnp.int32),
            pltpu.VMEM((n_chunks, CHUNK), jnp.int32),
            pltpu.VMEM((n_chunks * CHUNK,), jnp.float32),
            pltpu.VMEM((CHUNK, d_feat), jnp.float32),
            pltpu.VMEM_SHARED((n_nodes_pad, d_feat), jnp.float32),
        ],
    )
    def agg(g_hbm, src_hbm, dst_hbm, w_hbm, out_hbm, src_v, dst_v, w_v,
            rows_v, zacc):
        cid = lax.axis_index("c")
        sid = lax.axis_index("s")
        wid = cid * NS + sid

        def zero_row(r, _):
            for c in range(d_feat // LANES):
                rows_v[r, pl.ds(c * LANES, LANES)] = jnp.zeros(
                    (LANES,), jnp.float32)
            return _
        lax.fori_loop(0, CHUNK, zero_row, None)
        base = sid * rows_per_tile
        full = rows_per_tile // CHUNK
        for cidx in range(full):
            pltpu.sync_copy(rows_v,
                            zacc.at[pl.ds(base + cidx * CHUNK, CHUNK)])
        plsc.subcore_barrier()

        pltpu.sync_copy(src_hbm.at[wid], src_v)
        pltpu.sync_copy(dst_hbm.at[wid], dst_v)
        pltpu.sync_copy(w_hbm.at[wid], w_v)

        def chunk_body(j, _):
            pltpu.sync_copy(g_hbm.at[src_v.at[j]], rows_v)

            def scale_group(gi, _):
                w16 = w_v[pl.ds(j * CHUNK + gi * LANES, LANES)]
                for k in range(LANES):
                    e = gi * LANES + k
                    wsplat = _splat(w16, k)
                    for c in range(d_feat // LANES):
                        sl = pl.ds(c * LANES, LANES)
                        rows_v[e, sl] = rows_v[e, sl] * wsplat
                return _
            lax.fori_loop(0, CHUNK // LANES, scale_group, None)

            pltpu.sync_copy(rows_v, zacc.at[dst_v.at[j]], add=True)
            return _
        lax.fori_loop(0, n_chunks, chunk_body, None)

        plsc.subcore_barrier()

        for cidx in range(full):
            sl = pl.ds(base + cidx * CHUNK, CHUNK)
            pltpu.sync_copy(zacc.at[sl], rows_v)
            pltpu.sync_copy(rows_v, out_hbm.at[cid].at[sl])

    return agg


@jax.jit
def kernel(h, edge_index, w, d, W_w, W_b):
    n_nodes, d_feat = h.shape
    n_edges = w.shape[0]

    per_w = -(-n_edges // (NW * CHUNK)) * CHUNK
    e_pad = per_w * NW
    pad = e_pad - n_edges
    src = jnp.concatenate([edge_index[0], jnp.zeros((pad,), jnp.int32)])
    dst = jnp.concatenate([edge_index[1], jnp.zeros((pad,), jnp.int32)])
    w_p = jnp.concatenate([w, jnp.zeros((pad,), jnp.float32)])
    n_chunks = per_w // CHUNK
    src = src.reshape(NW, n_chunks, CHUNK)
    dst = dst.reshape(NW, n_chunks, CHUNK)
    w_p = w_p.reshape(NW, n_chunks * CHUNK)

    d2 = d.reshape(n_nodes, 1)
    b2 = W_b.reshape(1, d_feat)
    wt = W_w.T

    br = 2000
    grid = n_nodes // br
    g = pl.pallas_call(
        _matmul_body,
        grid=(grid,),
        in_specs=[
            pl.BlockSpec((br, d_feat), lambda i: (i, 0)),
            pl.BlockSpec((d_feat, d_feat), lambda i: (0, 0)),
            pl.BlockSpec((1, d_feat), lambda i: (0, 0)),
            pl.BlockSpec((br, 1), lambda i: (i, 0)),
        ],
        out_specs=pl.BlockSpec((br, d_feat), lambda i: (i, 0)),
        out_shape=jax.ShapeDtypeStruct((n_nodes, d_feat), jnp.float32),
    )(h, wt, b2, d2)

    n_nodes_pad = -(-n_nodes // (NS * CHUNK)) * NS * CHUNK
    partials = _sc_aggregate(n_nodes_pad, d_feat, n_chunks)(g, src, dst, w_p)

    z = pl.pallas_call(
        _combine_body,
        grid=(grid,),
        in_specs=[
            pl.BlockSpec((NC, br, d_feat), lambda i: (0, i, 0)),
            pl.BlockSpec((br, 1), lambda i: (i, 0)),
        ],
        out_specs=pl.BlockSpec((br, d_feat), lambda i: (i, 0)),
        out_shape=jax.ShapeDtypeStruct((n_nodes, d_feat), jnp.float32),
    )(partials, d2)
    return z
